# baseline (speedup 1.0000x reference)
import jax, jax.numpy as jnp
from jax.experimental import pallas as pl
from jax.experimental.pallas import tpu as pltpu

B = 16; NPG = 1024; N = 16384; E = 262144; EPG = 16384; K = 16


def _seg_softmax(logits, dst, n):
    m = jax.ops.segment_max(logits, dst, num_segments=n)
    m = jnp.where(jnp.isfinite(m), m, 0.0)
    e = jnp.exp(logits - m[dst])
    d = jax.ops.segment_sum(e, dst, num_segments=n)
    return e / (d[dst] + 1e-16)


def _egat(x, ei, ea, W, a_s, a_d, heads, od, concat):
    n = x.shape[0]
    src, dst = ei[0], ei[1]
    h = (x @ W).reshape(n, heads, od)
    hs = h[src]; hd = h[dst]
    logits = jax.nn.leaky_relu((hs * a_s[None]).sum(-1) + (hd * a_d[None]).sum(-1), 0.2)
    if ea is not None:
        logits = logits + ea[:, None]
    alpha = _seg_softmax(logits, dst, n)
    out = jax.ops.segment_sum(alpha[:, :, None] * hs, dst, num_segments=n)
    out = out.reshape(n, heads * od) if concat else out.mean(1)
    return out, alpha.mean(1), ei, ea


def _egat_coarse(h2, ea_mat, W, a_s, a_d):
    hh = h2 @ W
    ps = (hh * a_s[0]).sum(-1)
    pdn = (hh * a_d[0]).sum(-1)
    lg = jax.nn.leaky_relu(ps[:, :, None] + pdn[:, None, :], 0.2) + ea_mat
    m = lg.max(1, keepdims=True)
    e = jnp.exp(lg - m)
    Z = e.sum(1, keepdims=True)
    alpha = e / (Z + 1e-16)
    out = jnp.einsum('bij,bic->bjc', alpha, hh,
                     precision=jax.lax.Precision.HIGHEST)
    return out, alpha


def _inorm(x, batch, ng):
    cnt = jax.ops.segment_sum(jnp.ones((x.shape[0], 1), x.dtype), batch, num_segments=ng)
    mean = jax.ops.segment_sum(x, batch, num_segments=ng) / cnt
    var = jax.ops.segment_sum(x * x, batch, num_segments=ng) / cnt - mean ** 2
    var = jnp.maximum(var, 0.0)
    return (x - mean[batch]) / jnp.sqrt(var[batch] + 1e-5)


def _inorm1(x):
    m = x.mean(0, keepdims=True); v = x.var(0, keepdims=True)
    return (x - m) / jnp.sqrt(v + 1e-5)


def _pool_kernel(adj_ref, s_ref, sr_ref, padj_ref, adj2_ref):
    r = pl.program_id(1)

    @pl.when(r == 0)
    def _():
        padj_ref[...] = jnp.zeros_like(padj_ref)

    a = adj_ref[0]
    t = jnp.dot(a, s_ref[0], preferred_element_type=jnp.float32)
    padj_ref[0] += jnp.dot(sr_ref[0].T, t, preferred_element_type=jnp.float32)
    adj2_ref[0, 0, 0, 0] = jnp.sum(a * a)


def _pool_stats(adj, s):
    RB = 256
    nr = NPG // RB
    padj, adj2 = pl.pallas_call(
        _pool_kernel,
        grid=(B, nr),
        in_specs=[
            pl.BlockSpec((1, RB, NPG), lambda b, r: (b, r, 0)),
            pl.BlockSpec((1, NPG, K), lambda b, r: (b, 0, 0)),
            pl.BlockSpec((1, RB, K), lambda b, r: (b, r, 0)),
        ],
        out_specs=[
            pl.BlockSpec((1, K, K), lambda b, r: (b, 0, 0)),
            pl.BlockSpec((1, 1, 1, 1), lambda b, r: (b, r, 0, 0),
                         memory_space=pltpu.SMEM),
        ],
        out_shape=[
            jax.ShapeDtypeStruct((B, K, K), jnp.float32),
            jax.ShapeDtypeStruct((B, nr, 1, 1), jnp.float32),
        ],
    )(adj, s, s)
    return padj, jnp.sum(adj2)


def kernel(x, edge_index, edge_attr, batch_mask, params):
    p = params
    src, dst = edge_index[0], edge_index[1]
    gid = src // NPG
    adj = jnp.zeros((B, NPG, NPG), jnp.float32).at[gid, src % NPG, dst % NPG].add(edge_attr)
    out_all = []
    a, ei, h = edge_attr, edge_index, x
    for name, heads, od in [("c1_1", 5, 30), ("c1_2", 1, 30), ("c1_3", 1, 30)]:
        h, a, ei, _ = _egat(h, ei, a, p[name + "_W"], p[name + "_as"], p[name + "_ad"], heads, od, False)
        h = _inorm(h, batch_mask, B)
        out_all.append(h)
    a, ei, hp = edge_attr, edge_index, x
    pool_outs = []
    for name, od in [("p_1", 30), ("p_2", 30), ("p_3", 16)]:
        hp, a, ei, _ = _egat(hp, ei, a, p[name + "_W"], p[name + "_as"], p[name + "_ad"], 1, od, False)
        hp = _inorm(hp, batch_mask, B)
        pool_outs.append(hp)
    pc = jnp.concatenate(pool_outs, 1)
    t = jnp.maximum(_inorm1(pc @ p["pf_W1"] + p["pf_b1"]), 0.0)
    assign = (t @ p["pf_W2"] + p["pf_b2"]).reshape(B, NPG, K)
    s = jax.nn.softmax(assign, -1)
    xc = out_all[-1].reshape(B, NPG, 30)
    px = jnp.einsum('bnk,bnf->bkf', s, xc)
    padj = jnp.einsum('bnk,bnm,bmj->bkj', s, adj, s)
    _padj_pl, adj2 = _pool_stats(adj, s)
    sts = jnp.einsum('bnk,bnl->bkl', s, s,
                     precision=jax.lax.Precision.HIGHEST)
    tr = jnp.trace(padj, axis1=1, axis2=2).sum()
    g2 = jnp.sum(sts * sts)
    link = jnp.sqrt(jnp.maximum(adj2 - 2.0 * tr + g2, 0.0)) / (B * NPG * NPG)
    ent = (-s * jnp.log(s + 1e-15)).sum(-1).mean()
    reg = (link + ent).reshape(1)
    ii = jnp.arange(K)
    off = jnp.repeat(jnp.arange(B) * K, K * K)
    src2 = jnp.tile(jnp.repeat(ii, K), B) + off
    dst2 = jnp.tile(jnp.tile(ii, K), B) + off
    ei2 = jnp.stack([src2, dst2])
    ea2 = padj.reshape(-1)
    mask2 = jnp.arange(B * K) // K
    a2, ei_c, h2 = ea2, ei2, px.reshape(B * K, 30)
    for name in ["c2_1", "c2_2", "c2_3"]:
        h2, a2, ei_c, _ = _egat(h2, ei_c, a2, p[name + "_W"], p[name + "_as"],
                                p[name + "_ad"], 1, 30, False)
        h2 = _inorm(h2, mask2, B)
        out_all.append(h2)
    g = jnp.concatenate([o.reshape(B, -1, o.shape[-1]).max(1) for o in out_all], -1)
    t = jnp.maximum(_inorm1(g @ p["fc_W1"] + p["fc_b1"]), 0.0)
    fc_out = t @ p["fc_W2"] + p["fc_b2"]
    return fc_out, reg


# speedup vs baseline: 1.2836x; 1.2836x over previous
import functools

import jax, jax.numpy as jnp
from jax.experimental import pallas as pl
from jax.experimental.pallas import tpu as pltpu
from jax.experimental.pallas import tpu_sc as plsc

B = 16; NPG = 1024; N = 16384; E = 262144; EPG = 16384; K = 16


_NW = 32
_CH = 128


@functools.lru_cache(maxsize=None)
def _make_sc_gather(nrows, dp):
    per_w = nrows // _NW
    nit = per_w // _CH
    mesh = plsc.VectorSubcoreMesh(core_axis_name="c", subcore_axis_name="s")

    @functools.partial(
        pl.kernel, mesh=mesh,
        out_type=jax.ShapeDtypeStruct((nrows, dp), jnp.float32),
        scratch_types=[
            pltpu.VMEM((_CH,), jnp.int32),
            pltpu.VMEM((_CH, dp), jnp.float32),
            pltpu.SemaphoreType.DMA,
        ],
    )
    def k(table_hbm, idx_hbm, out_hbm, idx_v, rows_v, sem):
        wid = jax.lax.axis_index("s") * 2 + jax.lax.axis_index("c")
        base = wid * per_w

        @pl.loop(0, nit)
        def body(j):
            off = base + j * _CH
            pltpu.sync_copy(idx_hbm.at[pl.ds(off, _CH)], idx_v)
            pltpu.async_copy(table_hbm.at[idx_v], rows_v, sem).wait()
            pltpu.sync_copy(rows_v, out_hbm.at[pl.ds(off, _CH)])

    return k


def _sc_rows(table, idx, dp):
    n, d = table.shape
    tp = table if d == dp else jnp.pad(table, ((0, 0), (0, dp - d)))
    out = _make_sc_gather(idx.shape[0], dp)(tp, idx.astype(jnp.int32))
    return out[:, :d]


def _egat(x, ei, ea, W, a_s, a_d, heads, od, concat, use_sc=False):
    n = x.shape[0]
    src, dst = ei[0], ei[1]
    h = (x @ W).reshape(n, heads, od)
    hs = h[src]; hd = h[dst]
    logits = jax.nn.leaky_relu((hs * a_s[None]).sum(-1) + (hd * a_d[None]).sum(-1), 0.2)
    if ea is not None:
        logits = logits + ea[:, None]
    m = jax.ops.segment_max(logits, dst, num_segments=n)
    m = jnp.where(jnp.isfinite(m), m, 0.0)
    m_dst = _sc_rows(m, dst, 128) if use_sc else m[dst]
    e = jnp.exp(logits - m_dst)
    d = jax.ops.segment_sum(e, dst, num_segments=n)
    d_dst = _sc_rows(d, dst, 128) if use_sc else d[dst]
    alpha = e / (d_dst + 1e-16)
    out = jax.ops.segment_sum(alpha[:, :, None] * hs, dst, num_segments=n)
    out = out.reshape(n, heads * od) if concat else out.mean(1)
    return out, alpha.mean(1), ei, ea


def _egat_coarse(h2, ea_mat, W, a_s, a_d):
    hh = h2 @ W
    ps = (hh * a_s[0]).sum(-1)
    pdn = (hh * a_d[0]).sum(-1)
    lg = jax.nn.leaky_relu(ps[:, :, None] + pdn[:, None, :], 0.2) + ea_mat
    m = lg.max(1, keepdims=True)
    e = jnp.exp(lg - m)
    Z = e.sum(1, keepdims=True)
    alpha = e / (Z + 1e-16)
    out = jnp.einsum('bij,bic->bjc', alpha, hh,
                     precision=jax.lax.Precision.HIGHEST)
    return out, alpha


def _inorm(x, batch, ng):
    cnt = jax.ops.segment_sum(jnp.ones((x.shape[0], 1), x.dtype), batch, num_segments=ng)
    mean = jax.ops.segment_sum(x, batch, num_segments=ng) / cnt
    var = jax.ops.segment_sum(x * x, batch, num_segments=ng) / cnt - mean ** 2
    var = jnp.maximum(var, 0.0)
    return (x - mean[batch]) / jnp.sqrt(var[batch] + 1e-5)


def _inorm1(x):
    m = x.mean(0, keepdims=True); v = x.var(0, keepdims=True)
    return (x - m) / jnp.sqrt(v + 1e-5)


def _pool_kernel(adj_ref, s_ref, sr_ref, padj_ref, adj2_ref):
    r = pl.program_id(1)

    @pl.when(r == 0)
    def _():
        padj_ref[...] = jnp.zeros_like(padj_ref)

    a = adj_ref[0]
    t = jnp.dot(a, s_ref[0], preferred_element_type=jnp.float32)
    padj_ref[0] += jnp.dot(sr_ref[0].T, t, preferred_element_type=jnp.float32)
    adj2_ref[0, 0, 0, 0] = jnp.sum(a * a)


def _pool_stats(adj, s):
    RB = 256
    nr = NPG // RB
    padj, adj2 = pl.pallas_call(
        _pool_kernel,
        grid=(B, nr),
        in_specs=[
            pl.BlockSpec((1, RB, NPG), lambda b, r: (b, r, 0)),
            pl.BlockSpec((1, NPG, K), lambda b, r: (b, 0, 0)),
            pl.BlockSpec((1, RB, K), lambda b, r: (b, r, 0)),
        ],
        out_specs=[
            pl.BlockSpec((1, K, K), lambda b, r: (b, 0, 0)),
            pl.BlockSpec((1, 1, 1, 1), lambda b, r: (b, r, 0, 0),
                         memory_space=pltpu.SMEM),
        ],
        out_shape=[
            jax.ShapeDtypeStruct((B, K, K), jnp.float32),
            jax.ShapeDtypeStruct((B, nr, 1, 1), jnp.float32),
        ],
    )(adj, s, s)
    return padj, jnp.sum(adj2)


def kernel(x, edge_index, edge_attr, batch_mask, params):
    p = params
    src, dst = edge_index[0], edge_index[1]
    gid = src // NPG
    adj = jnp.zeros((B, NPG, NPG), jnp.float32).at[gid, src % NPG, dst % NPG].add(edge_attr)
    out_all = []
    a, ei, h = edge_attr, edge_index, x
    for name, heads, od in [("c1_1", 5, 30), ("c1_2", 1, 30), ("c1_3", 1, 30)]:
        h, a, ei, _ = _egat(h, ei, a, p[name + "_W"], p[name + "_as"], p[name + "_ad"], heads, od, False, use_sc=True)
        h = _inorm(h, batch_mask, B)
        out_all.append(h)
    a, ei, hp = edge_attr, edge_index, x
    pool_outs = []
    for name, od in [("p_1", 30), ("p_2", 30), ("p_3", 16)]:
        hp, a, ei, _ = _egat(hp, ei, a, p[name + "_W"], p[name + "_as"], p[name + "_ad"], 1, od, False, use_sc=True)
        hp = _inorm(hp, batch_mask, B)
        pool_outs.append(hp)
    pc = jnp.concatenate(pool_outs, 1)
    t = jnp.maximum(_inorm1(pc @ p["pf_W1"] + p["pf_b1"]), 0.0)
    assign = (t @ p["pf_W2"] + p["pf_b2"]).reshape(B, NPG, K)
    s = jax.nn.softmax(assign, -1)
    xc = out_all[-1].reshape(B, NPG, 30)
    px = jnp.einsum('bnk,bnf->bkf', s, xc)
    padj = jnp.einsum('bnk,bnm,bmj->bkj', s, adj, s)
    _padj_pl, adj2 = _pool_stats(adj, s)
    sts = jnp.einsum('bnk,bnl->bkl', s, s,
                     precision=jax.lax.Precision.HIGHEST)
    tr = jnp.trace(padj, axis1=1, axis2=2).sum()
    g2 = jnp.sum(sts * sts)
    link = jnp.sqrt(jnp.maximum(adj2 - 2.0 * tr + g2, 0.0)) / (B * NPG * NPG)
    ent = (-s * jnp.log(s + 1e-15)).sum(-1).mean()
    reg = (link + ent).reshape(1)
    ii = jnp.arange(K)
    off = jnp.repeat(jnp.arange(B) * K, K * K)
    src2 = jnp.tile(jnp.repeat(ii, K), B) + off
    dst2 = jnp.tile(jnp.tile(ii, K), B) + off
    ei2 = jnp.stack([src2, dst2])
    ea2 = padj.reshape(-1)
    mask2 = jnp.arange(B * K) // K
    a2, ei_c, h2 = ea2, ei2, px.reshape(B * K, 30)
    for name in ["c2_1", "c2_2", "c2_3"]:
        h2, a2, ei_c, _ = _egat(h2, ei_c, a2, p[name + "_W"], p[name + "_as"],
                                p[name + "_ad"], 1, 30, False)
        h2 = _inorm(h2, mask2, B)
        out_all.append(h2)
    g = jnp.concatenate([o.reshape(B, -1, o.shape[-1]).max(1) for o in out_all], -1)
    t = jnp.maximum(_inorm1(g @ p["fc_W1"] + p["fc_b1"]), 0.0)
    fc_out = t @ p["fc_W2"] + p["fc_b2"]
    return fc_out, reg
